# baseline (speedup 1.0000x reference)
import jax
import jax.numpy as jnp
from jax.experimental import pallas as pl
from jax.experimental.pallas import tpu as pltpu


def _fused_kernel(adj_ref, x_ref, w_ref, b_ref, out_ref, s_ref):
    @pl.when(pl.program_id(0) == 0)
    def _():
        s_ref[...] = jnp.dot(x_ref[...], w_ref[...],
                             preferred_element_type=jnp.float32)

    out_ref[...] = jnp.dot(adj_ref[...], s_ref[...],
                           preferred_element_type=jnp.float32) + b_ref[...]


def kernel(adj, x, weight, bias):
    n, k = adj.shape
    d_in, d_out = weight.shape
    bias2 = bias.reshape(1, d_out)

    bm = 80
    out = pl.pallas_call(
        _fused_kernel,
        grid=(n // bm,),
        in_specs=[
            pl.BlockSpec((bm, k), lambda i: (i, 0)),
            pl.BlockSpec((k, d_in), lambda i: (0, 0)),
            pl.BlockSpec((d_in, d_out), lambda i: (0, 0)),
            pl.BlockSpec((1, d_out), lambda i: (0, 0)),
        ],
        out_specs=pl.BlockSpec((bm, d_out), lambda i: (i, 0)),
        out_shape=jax.ShapeDtypeStruct((n, d_out), jnp.float32),
        scratch_shapes=[pltpu.VMEM((k, d_out), jnp.float32)],
    )(adj, x, weight, bias2)
    return out


# speedup vs baseline: 1.3682x; 1.3682x over previous
import jax
import jax.numpy as jnp
from jax.experimental import pallas as pl
from jax.experimental.pallas import tpu as pltpu


def _fused_kernel(adj_a_ref, adj_b_ref, x_ref, w_ref, b_ref, out_ref, s_ref):
    @pl.when(pl.program_id(0) == 0)
    def _():
        s_ref[...] = jnp.dot(x_ref[...], w_ref[...],
                             preferred_element_type=jnp.float32)

    h = adj_a_ref.shape[1]
    out_ref[:h, :] = jnp.dot(adj_a_ref[0], s_ref[...],
                             preferred_element_type=jnp.float32) + b_ref[...]
    out_ref[h:, :] = jnp.dot(adj_b_ref[0], s_ref[...],
                             preferred_element_type=jnp.float32) + b_ref[...]


def kernel(adj, x, weight, bias):
    n, k = adj.shape
    d_in, d_out = weight.shape
    bias2 = bias.reshape(1, d_out)

    half = 200
    adj3 = adj.reshape(n // half, half, k)

    out = pl.pallas_call(
        _fused_kernel,
        grid=(n // (2 * half),),
        in_specs=[
            pl.BlockSpec((1, half, k), lambda i: (2 * i, 0, 0)),
            pl.BlockSpec((1, half, k), lambda i: (2 * i + 1, 0, 0)),
            pl.BlockSpec((k, d_in), lambda i: (0, 0)),
            pl.BlockSpec((d_in, d_out), lambda i: (0, 0)),
            pl.BlockSpec((1, d_out), lambda i: (0, 0)),
        ],
        out_specs=pl.BlockSpec((2 * half, d_out), lambda i: (i, 0)),
        out_shape=jax.ShapeDtypeStruct((n, d_out), jnp.float32),
        scratch_shapes=[pltpu.VMEM((k, d_out), jnp.float32)],
    )(adj3, adj3, x, weight, bias2)
    return out
